# baseline (speedup 1.0000x reference)
import jax
import jax.numpy as jnp
from jax.experimental import pallas as pl

_EPS = 1e-8


_BM = 256


def _simrel_kernel(x_ref, ca_ref, out_ref):
    x = x_ref[...]
    ca = ca_ref[...]
    inv_in = 1.0 / jnp.maximum(jnp.sqrt(jnp.sum(x * x, axis=1, keepdims=True)), _EPS)
    inv_ca = 1.0 / jnp.maximum(jnp.sqrt(jnp.sum(ca * ca, axis=1)), _EPS)
    dots = jax.lax.dot_general(
        x, ca,
        dimension_numbers=(((1,), (1,)), ((), ())),
        preferred_element_type=jnp.float32,
    )
    out_ref[...] = dots * inv_in * inv_ca[None, :]


def kernel(inputs, labels, class_avgs):
    del labels
    b, s, d = inputs.shape
    k = class_avgs.shape[0]
    m = b * s
    x = inputs.reshape(m, d)
    out = pl.pallas_call(
        _simrel_kernel,
        grid=(m // _BM,),
        in_specs=[
            pl.BlockSpec((_BM, d), lambda i: (i, 0)),
            pl.BlockSpec((k, d), lambda i: (0, 0)),
        ],
        out_specs=pl.BlockSpec((_BM, k), lambda i: (i, 0)),
        out_shape=jax.ShapeDtypeStruct((m, k), jnp.float32),
    )(x, class_avgs)
    return out.reshape(b, s, k)


# speedup vs baseline: 1.2198x; 1.2198x over previous
import jax
import jax.numpy as jnp
from jax.experimental import pallas as pl

_EPS = 1e-8


def _simrel_kernel(x_ref, ca_ref, out_ref):
    x = x_ref[...]
    ca = ca_ref[...]
    inv_in = 1.0 / jnp.maximum(jnp.sqrt(jnp.sum(x * x, axis=1, keepdims=True)), _EPS)
    inv_ca = 1.0 / jnp.maximum(jnp.sqrt(jnp.sum(ca * ca, axis=1)), _EPS)
    dots = jax.lax.dot_general(
        x, ca,
        dimension_numbers=(((1,), (1,)), ((), ())),
        preferred_element_type=jnp.float32,
    )
    out_ref[...] = dots * inv_in * inv_ca[None, :]


def kernel(inputs, labels, class_avgs):
    del labels
    b, s, d = inputs.shape
    k = class_avgs.shape[0]
    m = b * s
    x = inputs.reshape(m, d)
    out = pl.pallas_call(
        _simrel_kernel,
        out_shape=jax.ShapeDtypeStruct((m, k), jnp.float32),
    )(x, class_avgs)
    return out.reshape(b, s, k)
